# baseline (speedup 1.0000x reference)
import functools

import jax
import jax.numpy as jnp
from jax import lax
from jax.experimental import pallas as pl
from jax.experimental.pallas import tpu as pltpu
from jax.experimental.pallas import tpu_sc as plsc

NUM_REL = 1000000
K = 16
BATCH = 16384

_NUM_CORES = 2
_NUM_SUBCORES = 16
_NW = _NUM_CORES * _NUM_SUBCORES
_B_PER_W = BATCH // _NW
_PHYS_COLS = 128
_ROWS_PER_PHYS = _PHYS_COLS // K
_PHYS_ROWS = NUM_REL // _ROWS_PER_PHYS
_FLAT_PER_W = _B_PER_W * K

_mesh = plsc.VectorSubcoreMesh(core_axis_name="c", subcore_axis_name="s")


@functools.partial(
    pl.kernel,
    mesh=_mesh,
    out_type=jax.ShapeDtypeStruct((BATCH * K,), jnp.float32),
    scratch_types=[
        pltpu.VMEM((_B_PER_W,), jnp.int32),
        pltpu.VMEM((_B_PER_W,), jnp.int32),
        pltpu.VMEM((_B_PER_W, _PHYS_COLS), jnp.float32),
        pltpu.VMEM((_FLAT_PER_W,), jnp.float32),
        pltpu.SemaphoreType.DMA,
    ],
    compiler_params=pltpu.CompilerParams(needs_layout_passes=False),
)
def _gather(idx_hbm, table_hbm, out_hbm, idx_v, pr_v, buf_v, rows1d, sem):
    wid = lax.axis_index("s") * _NUM_CORES + lax.axis_index("c")
    base = wid * _B_PER_W
    iota = lax.iota(jnp.int32, 16)
    pltpu.sync_copy(idx_hbm.at[pl.ds(base, _B_PER_W)], idx_v)

    def compute_pr(i, carry):
        v = idx_v[pl.ds(i * 16, 16)]
        pr_v[pl.ds(i * 16, 16)] = lax.shift_right_logical(v, 3)
        return carry

    lax.fori_loop(0, _B_PER_W // 16, compute_pr, 0)

    pltpu.async_copy(table_hbm.at[pr_v], buf_v, sem).wait()

    def extract(g, carry):
        idx16 = idx_v[pl.ds(g * 16, 16)]
        offs = (idx16 & 7) * K
        rows = g * 16 + iota
        fbase = rows * K
        for c in range(K):
            vals = plsc.load_gather(buf_v, [rows, offs + c])
            plsc.store_scatter(rows1d, [fbase + c], vals)
        return carry

    lax.fori_loop(0, _B_PER_W // 16, extract, 0)

    pltpu.sync_copy(rows1d, out_hbm.at[pl.ds(wid * _FLAT_PER_W, _FLAT_PER_W)])


def kernel(batch_relation, rel_attention):
    table128 = rel_attention.reshape(_PHYS_ROWS, _PHYS_COLS)
    out = _gather(batch_relation.astype(jnp.int32), table128)
    return out.reshape(BATCH, K)


# speedup vs baseline: 7.0919x; 7.0919x over previous
import functools

import jax
import jax.numpy as jnp
from jax import lax
from jax.experimental import pallas as pl
from jax.experimental.pallas import tpu as pltpu
from jax.experimental.pallas import tpu_sc as plsc

NUM_REL = 1000000
K = 16
BATCH = 16384

_NUM_CORES = 2
_NUM_SUBCORES = 16
_NW = _NUM_CORES * _NUM_SUBCORES
_B_PER_W = BATCH // _NW
_G = _B_PER_W // 16
_LINE = 128

_mesh = plsc.VectorSubcoreMesh(core_axis_name="c", subcore_axis_name="s")


@functools.partial(
    pl.kernel,
    mesh=_mesh,
    out_type=jax.ShapeDtypeStruct((K, BATCH), jnp.float32),
    scratch_types=[
        pltpu.VMEM((_B_PER_W,), jnp.int32),
        pltpu.VMEM((2, 16, K, _LINE), jnp.float32),
        pltpu.VMEM((K, _B_PER_W), jnp.float32),
        pltpu.SemaphoreType.DMA,
    ],
    compiler_params=pltpu.CompilerParams(needs_layout_passes=False),
)
def _gather(idx_hbm, table_hbm, out_hbm, idx_v, blk_v, stage_v, sem):
    wid = lax.axis_index("s") * _NUM_CORES + lax.axis_index("c")
    base = wid * _B_PER_W
    iota = lax.iota(jnp.int32, 16)
    pltpu.sync_copy(idx_hbm.at[pl.ds(base, _B_PER_W)], idx_v)

    def copies(g, ring):
        v16 = idx_v[pl.ds(g * 16, 16)]
        out = []
        for j in range(16):
            col_al = pl.multiple_of(((v16[j] >> 7) << 7), _LINE)
            out.append(
                pltpu.make_async_copy(
                    table_hbm.at[:, pl.ds(col_al, _LINE)],
                    blk_v.at[ring, j],
                    sem,
                )
            )
        return v16, out

    def issue_group(g, ring):
        _, cps = copies(g, ring)
        for cp in cps:
            cp.start()

    def extract_group(g, ring):
        v16, cps = copies(g, ring)
        for cp in cps:
            cp.wait()
        for j in range(16):
            rem = jnp.full((16,), v16[j] & (_LINE - 1), jnp.int32)
            vals = plsc.load_gather(blk_v.at[ring, j], [iota, rem])
            plsc.store_scatter(
                stage_v, [iota, jnp.full((16,), g * 16 + j, jnp.int32)], vals
            )

    issue_group(0, 0)

    def body(g, carry):
        issue_group(g + 1, (g + 1) & 1)
        extract_group(g, g & 1)
        return carry

    lax.fori_loop(0, _G - 1, body, 0)
    extract_group(_G - 1, (_G - 1) & 1)

    pltpu.sync_copy(stage_v, out_hbm.at[:, pl.ds(base, _B_PER_W)])


def kernel(batch_relation, rel_attention):
    out_phys = _gather(batch_relation.astype(jnp.int32), rel_attention.T)
    return out_phys.T
